# Initial kernel scaffold; baseline (speedup 1.0000x reference)
import jax
import jax.numpy as jnp
from jax.experimental import pallas as pl


def kernel(x, weights, indices, W1, b1, W2, b2):
    raise NotImplementedError("write your pallas kernel here")


# speedup vs baseline: 2.2325x; 2.2325x over previous
import jax
import jax.numpy as jnp
from jax.experimental import pallas as pl
from jax.experimental.pallas import tpu as pltpu


def _moe_kernel(idx_ref, w_ref, x_ref, W1_ref, b1_ref, W2_ref, b2_ref, out_ref):
    b = pl.program_id(0)
    top_k = idx_ref.shape[1]
    c_out, hw = out_ref.shape[1], out_ref.shape[2]
    xb = x_ref[0]
    acc = jnp.zeros((c_out, hw), jnp.float32)
    for k in range(top_k):
        e = idx_ref[b, k]
        w = w_ref[b, k]
        w1 = W1_ref[e]
        h = jnp.dot(w1, xb, preferred_element_type=jnp.float32) + b1_ref[e][:, None]
        h = jnp.maximum(h, 0.0)
        w2 = W2_ref[e]
        y = jnp.dot(w2, h, preferred_element_type=jnp.float32) + b2_ref[e][:, None]
        acc = acc + w * y
    out_ref[0] = acc


def kernel(x, weights, indices, W1, b1, W2, b2):
    B, C_IN, H, W_SP = x.shape
    E, HIDDEN, _ = W1.shape
    C_OUT = W2.shape[1]
    HW = H * W_SP
    x3 = x.reshape(B, C_IN, HW)

    grid_spec = pltpu.PrefetchScalarGridSpec(
        num_scalar_prefetch=2,
        grid=(B,),
        in_specs=[
            pl.BlockSpec((1, C_IN, HW), lambda b, idx, w: (b, 0, 0)),
            pl.BlockSpec((E, HIDDEN, C_IN), lambda b, idx, w: (0, 0, 0)),
            pl.BlockSpec((E, HIDDEN), lambda b, idx, w: (0, 0)),
            pl.BlockSpec((E, C_OUT, HIDDEN), lambda b, idx, w: (0, 0, 0)),
            pl.BlockSpec((E, C_OUT), lambda b, idx, w: (0, 0)),
        ],
        out_specs=pl.BlockSpec((1, C_OUT, HW), lambda b, idx, w: (b, 0, 0)),
    )
    out = pl.pallas_call(
        _moe_kernel,
        grid_spec=grid_spec,
        out_shape=jax.ShapeDtypeStruct((B, C_OUT, HW), jnp.float32),
    )(indices, weights, x3, W1, b1, W2, b2)
    return out.reshape(B, C_OUT, H, W_SP)
